# Initial kernel scaffold; baseline (speedup 1.0000x reference)
import jax
import jax.numpy as jnp
from jax.experimental import pallas as pl


def kernel(embedding, x):
    raise NotImplementedError("write your pallas kernel here")


# speedup vs baseline: 2.9458x; 2.9458x over previous
import functools

import jax
import jax.numpy as jnp
from jax import lax
from jax.experimental import pallas as pl
from jax.experimental.pallas import tpu as pltpu
from jax.experimental.pallas import tpu_sc as plsc

PAD_TOKEN_ID = 0

_info = plsc.get_sparse_core_info()
_NC, _NS = _info.num_cores, _info.num_subcores
_NW = _NC * _NS

_B = 4096 * 50
_D = 128
_CHUNK = 128
_B_PER_W = _B // _NW
_NCHUNK = _B_PER_W // _CHUNK


def _make_kernel():
    mesh = plsc.VectorSubcoreMesh(core_axis_name="c", subcore_axis_name="s")

    @functools.partial(
        pl.kernel,
        mesh=mesh,
        out_type=jax.ShapeDtypeStruct((_B, _D), jnp.float32),
        scratch_types=[
            pltpu.VMEM((_NCHUNK, _CHUNK), jnp.int32),
            pltpu.VMEM((_CHUNK, _D), jnp.float32),
            pltpu.SemaphoreType.DMA,
        ],
    )
    def emb_kernel(table_hbm, x_hbm, out_hbm, idx_v, rows_v, sem):
        wid = lax.axis_index("s") * _NC + lax.axis_index("c")
        base = wid * _B_PER_W

        pltpu.sync_copy(x_hbm.at[wid], idx_v)

        def chunk_body(j, carry):
            pltpu.async_copy(table_hbm.at[idx_v.at[j]], rows_v, sem).wait()

            m = idx_v[j, pl.ds(0, 16)]
            for g in range(1, _CHUNK // 16):
                m = jnp.minimum(m, idx_v[j, pl.ds(g * 16, 16)])
            s = m[0]
            for lane in range(1, 16):
                s = jnp.minimum(s, m[lane])
            has_pad = s == PAD_TOKEN_ID

            @pl.when(has_pad)
            def _fix():
                z = jnp.zeros((16,), jnp.float32)

                def fix_group(g, c2):
                    vec = idx_v[j, pl.ds(g * 16, 16)]

                    rbase = g * 16
                    for lane in range(16):
                        @pl.when(vec[lane] == PAD_TOKEN_ID)
                        def _zero(lane=lane):
                            for cb in range(_D // 16):
                                rows_v[rbase + lane, pl.ds(cb * 16, 16)] = z
                    return c2

                lax.fori_loop(0, _CHUNK // 16, fix_group, 0)

            pltpu.sync_copy(rows_v, out_hbm.at[pl.ds(base + j * _CHUNK, _CHUNK)])
            return carry

        lax.fori_loop(0, _NCHUNK, chunk_body, 0)

    return emb_kernel


_emb_kernel = _make_kernel()


@jax.jit
def kernel(embedding, x):
    xs = x.reshape(-1).astype(jnp.int32).reshape(_NW, _NCHUNK, _CHUNK)
    out = _emb_kernel(embedding, xs)
    return out.reshape(x.shape[0], x.shape[1], _D)


# speedup vs baseline: 3.3308x; 1.1307x over previous
import functools

import jax
import jax.numpy as jnp
from jax import lax
from jax.experimental import pallas as pl
from jax.experimental.pallas import tpu as pltpu
from jax.experimental.pallas import tpu_sc as plsc

PAD_TOKEN_ID = 0

_info = plsc.get_sparse_core_info()
_NC, _NS = _info.num_cores, _info.num_subcores
_NW = _NC * _NS

_B = 4096 * 50
_D = 128
_CHUNK = 128
_B_PER_W = _B // _NW
_NCHUNK = _B_PER_W // _CHUNK
_RING = 5
_NOUTER = _NCHUNK // _RING


def _make_kernel():
    mesh = plsc.VectorSubcoreMesh(core_axis_name="c", subcore_axis_name="s")

    scratch = [pltpu.VMEM((_NCHUNK, _CHUNK), jnp.int32)]
    scratch += [pltpu.VMEM((_CHUNK, _D), jnp.float32) for _ in range(_RING)]
    scratch += [pltpu.SemaphoreType.DMA for _ in range(_RING)]

    @functools.partial(
        pl.kernel,
        mesh=mesh,
        out_type=jax.ShapeDtypeStruct((_B, _D), jnp.float32),
        scratch_types=scratch,
    )
    def emb_kernel(table_hbm, x_hbm, out_hbm, idx_v, *bufs_and_sems):
        bufs = bufs_and_sems[:_RING]
        sems = bufs_and_sems[_RING:]
        wid = lax.axis_index("s") * _NC + lax.axis_index("c")
        base = wid * _B_PER_W

        pltpu.sync_copy(x_hbm.at[wid], idx_v)

        def gather(j, b):
            pltpu.async_copy(table_hbm.at[idx_v.at[j]], bufs[b], sems[b])

        def wait(j, b):
            pltpu.make_async_copy(
                table_hbm.at[idx_v.at[j]], bufs[b], sems[b]
            ).wait()

        def fix_pads(j, b):
            m = idx_v[j, pl.ds(0, 16)]
            for g in range(1, _CHUNK // 16):
                m = jnp.minimum(m, idx_v[j, pl.ds(g * 16, 16)])
            s = m[0]
            for lane in range(1, 16):
                s = jnp.minimum(s, m[lane])

            @pl.when(s == PAD_TOKEN_ID)
            def _fix():
                z = jnp.zeros((16,), jnp.float32)

                def fix_group(g, c2):
                    vec = idx_v[j, pl.ds(g * 16, 16)]
                    rbase = g * 16
                    for lane in range(16):
                        @pl.when(vec[lane] == PAD_TOKEN_ID)
                        def _zero(lane=lane):
                            for cb in range(_D // 16):
                                bufs[b][rbase + lane, pl.ds(cb * 16, 16)] = z
                    return c2

                lax.fori_loop(0, _CHUNK // 16, fix_group, 0)

        for b in range(_RING):
            gather(b, b)

        def outer(t, carry):
            for b in range(_RING):
                j = t * _RING + b
                wait(j, b)
                fix_pads(j, b)
                pltpu.sync_copy(
                    bufs[b], out_hbm.at[pl.ds(base + j * _CHUNK, _CHUNK)]
                )

                @pl.when(t < _NOUTER - 1)
                def _next(j=j, b=b):
                    gather(j + _RING, b)
            return carry

        lax.fori_loop(0, _NOUTER, outer, 0)

    return emb_kernel


_emb_kernel = _make_kernel()


@jax.jit
def kernel(embedding, x):
    xs = x.reshape(-1).astype(jnp.int32).reshape(_NW, _NCHUNK, _CHUNK)
    out = _emb_kernel(embedding, xs)
    return out.reshape(x.shape[0], x.shape[1], _D)
